# baseline (speedup 1.0000x reference)
import jax
import jax.numpy as jnp
from jax.experimental import pallas as pl

B = 16
NUM_SLOTS = 32768
SLOT_DIM = 64
N_HEADS = 8
TOPK = 16
EPS = 1e-12


def _main_body(mem_ref, keys_ref, beta_ref, w_ref, read_ref):
    m = mem_ref[0]
    k = keys_ref[0]
    beta = beta_ref[0]

    kq = jnp.sqrt(jnp.sum(k * k, axis=1, keepdims=True))
    kn = k / jnp.maximum(kq, EPS)
    mq = jnp.sqrt(jnp.sum(m * m, axis=1, keepdims=True))
    mn = m / jnp.maximum(mq, EPS)

    sim = jax.lax.dot_general(
        kn, mn, (((1,), (1,)), ((), ())))
    sim = sim * beta.reshape(N_HEADS, 1)

    iota = jax.lax.broadcasted_iota(jnp.int32, (N_HEADS, NUM_SLOTS), 1)
    cur = sim
    selmask = jnp.zeros((N_HEADS, NUM_SLOTS), dtype=jnp.bool_)
    m0 = None
    for i in range(TOPK):
        mval = jnp.max(cur, axis=1, keepdims=True)
        if i == 0:
            m0 = mval
        sel_idx = jnp.min(
            jnp.where(cur == mval, iota, jnp.int32(NUM_SLOTS)),
            axis=1, keepdims=True)
        hit = iota == sel_idx
        selmask = jnp.logical_or(selmask, hit)
        cur = jnp.where(hit, -jnp.inf, cur)

    e = jnp.where(selmask, jnp.exp(sim - m0), 0.0)
    z = jnp.sum(e, axis=1, keepdims=True)
    w = e / z
    w_ref[0] = w

    read = jax.lax.dot_general(
        w, m, (((1,), (0,)), ((), ())),
        precision=jax.lax.Precision.HIGHEST)
    read_ref[0] = read


def _merge_body(read_ref, wm_ref, bm_ref, out_ref):
    out_ref[...] = jax.lax.dot_general(
        read_ref[...], wm_ref[...], (((1,), (1,)), ((), ())),
        precision=jax.lax.Precision.HIGHEST) + bm_ref[...]


def kernel(memory, read_keys, beta, W_merge, b_merge):
    beta3 = beta.reshape(B, 1, N_HEADS)

    weights, read_ph = pl.pallas_call(
        _main_body,
        grid=(B,),
        in_specs=[
            pl.BlockSpec((1, NUM_SLOTS, SLOT_DIM), lambda b: (b, 0, 0)),
            pl.BlockSpec((1, N_HEADS, SLOT_DIM), lambda b: (b, 0, 0)),
            pl.BlockSpec((1, 1, N_HEADS), lambda b: (b, 0, 0)),
        ],
        out_specs=[
            pl.BlockSpec((1, N_HEADS, NUM_SLOTS), lambda b: (b, 0, 0)),
            pl.BlockSpec((1, N_HEADS, SLOT_DIM), lambda b: (b, 0, 0)),
        ],
        out_shape=[
            jax.ShapeDtypeStruct((B, N_HEADS, NUM_SLOTS), jnp.float32),
            jax.ShapeDtypeStruct((B, N_HEADS, SLOT_DIM), jnp.float32),
        ],
    )(memory, read_keys, beta3)

    read_flat = read_ph.reshape(B, N_HEADS * SLOT_DIM)
    read_combined = pl.pallas_call(
        _merge_body,
        in_specs=[
            pl.BlockSpec(read_flat.shape, lambda: (0, 0)),
            pl.BlockSpec(W_merge.shape, lambda: (0, 0)),
            pl.BlockSpec((1, SLOT_DIM), lambda: (0, 0)),
        ],
        out_specs=pl.BlockSpec((B, SLOT_DIM), lambda: (0, 0)),
        out_shape=jax.ShapeDtypeStruct((B, SLOT_DIM), jnp.float32),
    )(read_flat, W_merge, b_merge.reshape(1, SLOT_DIM))

    return (read_combined, weights)
